# Initial kernel scaffold; baseline (speedup 1.0000x reference)
import jax
import jax.numpy as jnp
from jax.experimental import pallas as pl


def kernel(pos_u, pos_w, neg_w, W_word, W_ctx):
    raise NotImplementedError("write your pallas kernel here")


# speedup vs baseline: 1.9583x; 1.9583x over previous
import functools

import jax
import jax.numpy as jnp
from jax import lax
from jax.experimental import pallas as pl
from jax.experimental.pallas import tpu as pltpu
from jax.experimental.pallas import tpu_sc as plsc

_DIM = 64
_BATCH = 16384
_NNEG = 20
_NPAIR = _NNEG + 1
_NW = 32
_CH = 128
_NB = 4

_U_PER_W = _BATCH // _NW
_C_PER_W = _NPAIR * _BATCH // _NW


def _gather_loop(table, idx_hbm, out_hbm, base, ngroups, idx_bufs, row_bufs,
                 isems, osems):
    def group(g, carry):
        gathers = []
        for b in range(_NB):
            off = base + (g * _NB + b) * _CH
            pltpu.sync_copy(idx_hbm.at[pl.ds(off, _CH)], idx_bufs[b])
            gathers.append(pltpu.async_copy(table.at[idx_bufs[b]], row_bufs[b],
                                            isems[b]))
        outs = []
        for b in range(_NB):
            gathers[b].wait()
            off = base + (g * _NB + b) * _CH
            outs.append(pltpu.async_copy(row_bufs[b],
                                         out_hbm.at[pl.ds(off, _CH)], osems[b]))
        for b in range(_NB):
            outs[b].wait()
        return carry

    lax.fori_loop(0, ngroups, group, 0)


def _sc_gather_body(w_word, w_ctx, idx_u, idx_c, out_u, out_c, *scratch):
    idx_bufs = scratch[0:_NB]
    row_bufs = scratch[_NB:2 * _NB]
    isems = scratch[2 * _NB:3 * _NB]
    osems = scratch[3 * _NB:4 * _NB]
    wid = lax.axis_index("s") * 2 + lax.axis_index("c")
    _gather_loop(w_word, idx_u, out_u, wid * _U_PER_W, _U_PER_W // (_NB * _CH),
                 idx_bufs, row_bufs, isems, osems)
    _gather_loop(w_ctx, idx_c, out_c, wid * _C_PER_W, _C_PER_W // (_NB * _CH),
                 idx_bufs, row_bufs, isems, osems)


@functools.cache
def _sc_gather():
    return pl.kernel(
        _sc_gather_body,
        out_type=(
            jax.ShapeDtypeStruct((_BATCH, 2 * _DIM), jnp.float32),
            jax.ShapeDtypeStruct((_NPAIR * _BATCH, 2 * _DIM), jnp.float32),
        ),
        mesh=plsc.VectorSubcoreMesh(core_axis_name="c", subcore_axis_name="s"),
        scratch_types=(
            [pltpu.VMEM((_CH,), jnp.int32) for _ in range(_NB)]
            + [pltpu.VMEM((_CH, 2 * _DIM), jnp.float32) for _ in range(_NB)]
            + [pltpu.SemaphoreType.DMA for _ in range(2 * _NB)]
        ),
    )


def _logvol(z, Z):
    return jnp.sum(jnp.log(jax.nn.softplus(Z - z) + 1e-23), axis=-1)


def _tc_body(u_ref, c_ref, vols_ref, ints_ref, tv_ref):
    j = pl.program_id(1)
    u = u_ref[...]
    zu = jax.nn.sigmoid(u[:, :_DIM])
    Zu = zu + jax.nn.sigmoid(u[:, _DIM:]) * (1.0 - zu)
    c = c_ref[...]
    zc = jax.nn.sigmoid(c[:, :_DIM])
    Zc = zc + jax.nn.sigmoid(c[:, _DIM:]) * (1.0 - zc)
    vols_ref[0, 0, :] = _logvol(zc, Zc)
    zi = jnp.maximum(zc, zu)
    Zi = jnp.minimum(Zc, Zu)
    ints_ref[0, 0, :] = _logvol(zi, Zi)

    @pl.when(j == 0)
    def _():
        tv_ref[0, 0, :] = _logvol(zu, Zu)


def _tc_compute(u_rows, ctx_rows, bb=512):
    nb = _BATCH // bb
    return pl.pallas_call(
        _tc_body,
        grid=(nb, _NPAIR),
        in_specs=[
            pl.BlockSpec((bb, 2 * _DIM), lambda i, j: (i, 0)),
            pl.BlockSpec((bb, 2 * _DIM), lambda i, j, nb=nb: (j * nb + i, 0)),
        ],
        out_specs=[
            pl.BlockSpec((1, 1, bb), lambda i, j: (j, 0, i)),
            pl.BlockSpec((1, 1, bb), lambda i, j: (j, 0, i)),
            pl.BlockSpec((1, 1, bb), lambda i, j: (0, 0, i)),
        ],
        out_shape=[
            jax.ShapeDtypeStruct((_NPAIR, 1, _BATCH), jnp.float32),
            jax.ShapeDtypeStruct((_NPAIR, 1, _BATCH), jnp.float32),
            jax.ShapeDtypeStruct((1, 1, _BATCH), jnp.float32),
        ],
    )(u_rows, ctx_rows)


def kernel(pos_u, pos_w, neg_w, W_word, W_ctx):
    pos_u = pos_u.astype(jnp.int32)
    idx_ctx = jnp.concatenate(
        [neg_w.astype(jnp.int32).T.reshape(-1), pos_w.astype(jnp.int32)])
    u_rows, ctx_rows = _sc_gather()(W_word, W_ctx, pos_u, idx_ctx)
    vols, ints, tv = _tc_compute(u_rows, ctx_rows)
    vols, ints, tv = vols[:, 0, :], ints[:, 0, :], tv[:, 0, :]
    return (tv[0], vols[_NNEG], vols[:_NNEG].T, ints[_NNEG], ints[:_NNEG].T)
